# baseline (speedup 1.0000x reference)
import jax
import jax.numpy as jnp
from jax import lax
from jax.experimental import pallas as pl
from jax.experimental.pallas import tpu as pltpu
from jax.experimental.pallas import tpu_sc as plsc

B = 128
C = 10
N = B * C
W = 16384
ENC = 12288
CODE = 128
T0_ROWS = ENC + CODE
T_ROWS = W + CODE
TAU = 100.0

NCORES = 2
NSUB = 16
NW = NCORES * NSUB
ROWS_PER_TILE = W // NW
K = 16
CHUNKS = ROWS_PER_TILE // K
CODE_PER_TILE = CODE // NW


def _class_indicator():
    ci = lax.broadcasted_iota(jnp.int32, (C, N), 0)
    ni = lax.broadcasted_iota(jnp.int32, (C, N), 1)
    return (ni // B == ci).astype(jnp.float32)


_P1_PIX = 512
_P1_ROWS = 4 * _P1_PIX
_P1_ENC_BLOCKS = ENC // _P1_ROWS


def _p1_body(xt_ref, c0_ref, out_ref):
    i = pl.program_id(0)

    @pl.when(i < _P1_ENC_BLOCKS)
    def _():
        xb = xt_ref[...]
        bits = [(xb > (t + 1) / 5.0).astype(jnp.float32) for t in range(4)]
        bits = jnp.stack(bits, axis=1).reshape(_P1_ROWS, B)
        out_ref[...] = pltpu.repeat(bits, C, axis=1)

    @pl.when(i == _P1_ENC_BLOCKS)
    def _():
        code = 0.5 + 0.5 * jnp.sin(c0_ref[...])
        out_ref[0:CODE] = jnp.dot(code, _class_indicator(),
                                  preferred_element_type=jnp.float32)
        out_ref[CODE:] = jnp.zeros((_P1_ROWS - CODE, N), jnp.float32)


def _build_t0(x_t, ccl0_t):
    return pl.pallas_call(
        _p1_body,
        grid=(_P1_ENC_BLOCKS + 1,),
        in_specs=[
            pl.BlockSpec((_P1_PIX, B),
                         lambda i: (jnp.minimum(i, _P1_ENC_BLOCKS - 1), 0)),
            pl.BlockSpec((CODE, C), lambda i: (0, 0)),
        ],
        out_specs=pl.BlockSpec((_P1_ROWS, N), lambda i: (i, 0)),
        out_shape=jax.ShapeDtypeStruct((T0_ROWS, N), jnp.float32),
    )(x_t, ccl0_t)


_P2_BLK = 1024
_P2_GRID = W // _P2_BLK


def _p2_body(lg_ref, c0_ref, c1_ref, c2_ref):
    om = 0.5 + 0.5 * jnp.sin(lg_ref[...])
    ji = lax.broadcasted_iota(jnp.int32, (4, 64), 0)
    ci = lax.broadcasted_iota(jnp.int32, (4, 64), 1)
    sel = (ci // 16 == ji).astype(jnp.float32)
    for li, out_ref in enumerate((c0_ref, c1_ref, c2_ref)):
        w00 = om[li, 0:1, :]
        w01 = om[li, 1:2, :]
        w10 = om[li, 2:3, :]
        w11 = om[li, 3:4, :]
        terms = jnp.concatenate(
            [w00, w10 - w00, w01 - w00, w11 - w10 - w01 + w00], axis=0)
        out_ref[...] = lax.dot_general(
            terms, sel, (((0,), (0,)), ((), ())),
            preferred_element_type=jnp.float32)


def _build_coefs(logits_t):
    return pl.pallas_call(
        _p2_body,
        grid=(_P2_GRID,),
        in_specs=[pl.BlockSpec((3, 4, _P2_BLK), lambda i: (0, 0, i))],
        out_specs=[pl.BlockSpec((_P2_BLK, 64), lambda i: (i, 0))] * 3,
        out_shape=[jax.ShapeDtypeStruct((W, 64), jnp.float32)] * 3,
    )(logits_t)


def _p3_body(cc_ref, codes_ref):
    code = 0.5 + 0.5 * jnp.sin(cc_ref[...])
    ind = _class_indicator()
    for j in range(2):
        codes_ref[j] = jnp.dot(code[j], ind,
                               preferred_element_type=jnp.float32)


def _build_codes(ccl12_t):
    return pl.pallas_call(
        _p3_body,
        out_shape=jax.ShapeDtypeStruct((2, CODE, N), jnp.float32),
    )(ccl12_t)


_P4_BLK = 2048
_P4_GRID = W // _P4_BLK


def _p4_body(v_ref, out_ref):
    i = pl.program_id(0)
    s = jnp.sum(v_ref[...], axis=0, keepdims=True)

    @pl.when(i == 0)
    def _():
        out_ref[...] = s

    @pl.when(i > 0)
    def _():
        out_ref[...] += s


def _column_sum(v3):
    return pl.pallas_call(
        _p4_body,
        grid=(_P4_GRID,),
        in_specs=[pl.BlockSpec((_P4_BLK, N), lambda i: (i, 0))],
        out_specs=pl.BlockSpec((1, N), lambda i: (0, 0)),
        out_shape=jax.ShapeDtypeStruct((1, N), jnp.float32),
    )(v3)


_MESH = plsc.VectorSubcoreMesh(core_axis_name="c", subcore_axis_name="s",
                               num_cores=NCORES, num_subcores=NSUB)


def _base_scratch(extra):
    return [
        pltpu.VMEM((ROWS_PER_TILE,), jnp.int32),
        pltpu.VMEM((ROWS_PER_TILE,), jnp.int32),
        pltpu.VMEM((2, K, 64), jnp.float32),
        pltpu.VMEM((2, K, N), jnp.float32),
        pltpu.VMEM((2, K, N), jnp.float32),
        pltpu.VMEM((2, K, N), jnp.float32),
    ] + extra + [
        pltpu.SemaphoreType.DMA,
        pltpu.SemaphoreType.DMA,
        pltpu.SemaphoreType.DMA,
        pltpu.SemaphoreType.DMA,
        pltpu.SemaphoreType.DMA,
        pltpu.SemaphoreType.DMA,
        pltpu.SemaphoreType.DMA,
        pltpu.SemaphoreType.DMA,
    ]


def _worker_id():
    return lax.axis_index("s") * NCORES + lax.axis_index("c")


def _gathers(ci, b, base0, table, coef, lidx_v, ridx_v, coef_v, l_v, r_v,
             sgl, sgr, scf):
    loc = pl.ds(ci * K, K)
    return (
        pltpu.make_async_copy(table.at[lidx_v.at[loc]], l_v.at[b], sgl[b]),
        pltpu.make_async_copy(table.at[ridx_v.at[loc]], r_v.at[b], sgr[b]),
        pltpu.make_async_copy(coef.at[pl.ds(base0 + ci * K, K)],
                              coef_v.at[b], scf[b]),
    )


def _layer_body(table, lidx, ridx, coef, codes, out,
                lidx_v, ridx_v, coef_v, l_v, r_v, o_v,
                sgl0, sgl1, sgr0, sgr1, scf0, scf1, sst0, sst1):
    wid = _worker_id()
    base0 = wid * ROWS_PER_TILE
    sgl, sgr, scf, sst = (sgl0, sgl1), (sgr0, sgr1), (scf0, scf1), (sst0, sst1)
    pltpu.sync_copy(lidx.at[pl.ds(base0, ROWS_PER_TILE)], lidx_v)
    pltpu.sync_copy(ridx.at[pl.ds(base0, ROWS_PER_TILE)], ridx_v)
    for b in range(2):
        for desc in _gathers(b, b, base0, table, coef, lidx_v, ridx_v,
                             coef_v, l_v, r_v, sgl, sgr, scf):
            desc.start()

    def outer(i, carry):
        for b in range(2):
            ci = 2 * i + b
            for desc in _gathers(ci, b, base0, table, coef, lidx_v, ridx_v,
                                 coef_v, l_v, r_v, sgl, sgr, scf):
                desc.wait()

            @pl.when(i > 0)
            def _():
                pltpu.make_async_copy(
                    o_v.at[b], out.at[pl.ds(base0, K)], sst[b]).wait()

            def row(r, carry2):
                a, b_, c, d = _coef_vecs(coef_v.at[b], r)
                for cb in range(N // 16):
                    sl = pl.ds(cb * 16, 16)
                    l = l_v[b, r, sl]
                    rr = r_v[b, r, sl]
                    o_v[b, r, sl] = a + b_ * l + c * rr + d * (l * rr)
                return carry2

            lax.fori_loop(0, K, row, 0)
            pltpu.async_copy(o_v.at[b],
                             out.at[pl.ds(base0 + ci * K, K)], sst[b])

            @pl.when(ci + 2 < CHUNKS)
            def _():
                for desc in _gathers(ci + 2, b, base0, table, coef, lidx_v,
                                     ridx_v, coef_v, l_v, r_v, sgl, sgr, scf):
                    desc.start()
        return carry

    lax.fori_loop(0, CHUNKS // 2, outer, 0)
    for b in range(2):
        pltpu.make_async_copy(
            o_v.at[b], out.at[pl.ds(base0, K)], sst[b]).wait()

    if codes is not None:
        cb = wid * CODE_PER_TILE
        stage = l_v.at[0, pl.ds(0, CODE_PER_TILE)]
        pltpu.sync_copy(codes.at[pl.ds(cb, CODE_PER_TILE)], stage)
        pltpu.sync_copy(stage, out.at[pl.ds(W + cb, CODE_PER_TILE)])


def _coef_vecs(coef_vb, r):
    return [coef_vb[r, pl.ds(16 * j, 16)] for j in range(4)]


def _mid_layer(table, lidx, ridx, coef, codes):
    def body(table, lidx, ridx, coef, codes, out, *scratch):
        _layer_body(table, lidx, ridx, coef, codes, out, *scratch)

    return pl.kernel(
        body,
        out_type=jax.ShapeDtypeStruct((T_ROWS, N), jnp.float32),
        mesh=_MESH,
        scratch_types=_base_scratch([]),
    )(table, lidx, ridx, coef, codes)


def _final_layer(table, lidx, ridx, coef):
    def body(table, lidx, ridx, coef, out, *scratch):
        _layer_body(table, lidx, ridx, coef, None, out, *scratch)

    return pl.kernel(
        body,
        out_type=jax.ShapeDtypeStruct((W, N), jnp.float32),
        mesh=_MESH,
        scratch_types=_base_scratch([]),
    )(table, lidx, ridx, coef)


def kernel(x, logits0, logits1, logits2, class_code_logits,
           left0, right0, left1, right1, left2, right2):
    x_t = x.reshape(B, -1).T
    ccl0_t = class_code_logits[0].T
    ccl12_t = jnp.transpose(class_code_logits[1:], (0, 2, 1))
    logits_t = jnp.transpose(
        jnp.stack([logits0, logits1, logits2]), (0, 2, 1))

    t0 = _build_t0(x_t, ccl0_t)
    coef0, coef1, coef2 = _build_coefs(logits_t)
    codes = _build_codes(ccl12_t)

    t1 = _mid_layer(t0, left0, right0, coef0, codes[0])
    t2 = _mid_layer(t1, left1, right1, coef1, codes[1])
    v3 = _final_layer(t2, left2, right2, coef2)
    sums = _column_sum(v3)

    res = sums[0] / TAU
    return res.reshape(C, B).T


# speedup vs baseline: 1.2037x; 1.2037x over previous
import jax
import jax.numpy as jnp
from jax import lax
from jax.experimental import pallas as pl
from jax.experimental.pallas import tpu as pltpu
from jax.experimental.pallas import tpu_sc as plsc

B = 128
C = 10
N = B * C
W = 16384
ENC = 12288
CODE = 128
T0_ROWS = ENC + CODE
T_ROWS = W + CODE
TAU = 100.0

NCORES = 2
NSUB = 16
NW = NCORES * NSUB
ROWS_PER_TILE = W // NW
K = 32
CHUNKS = ROWS_PER_TILE // K
CODE_PER_TILE = CODE // NW


def _class_indicator():
    ci = lax.broadcasted_iota(jnp.int32, (C, N), 0)
    ni = lax.broadcasted_iota(jnp.int32, (C, N), 1)
    return (ni // B == ci).astype(jnp.float32)


_P1_PIX = 512
_P1_ROWS = 4 * _P1_PIX
_P1_ENC_BLOCKS = ENC // _P1_ROWS


def _p1_body(xt_ref, c0_ref, out_ref):
    i = pl.program_id(0)

    @pl.when(i < _P1_ENC_BLOCKS)
    def _():
        xb = xt_ref[...]
        bits = [(xb > (t + 1) / 5.0).astype(jnp.float32) for t in range(4)]
        bits = jnp.stack(bits, axis=1).reshape(_P1_ROWS, B)
        out_ref[...] = pltpu.repeat(bits, C, axis=1)

    @pl.when(i == _P1_ENC_BLOCKS)
    def _():
        code = 0.5 + 0.5 * jnp.sin(c0_ref[...])
        out_ref[0:CODE] = jnp.dot(code, _class_indicator(),
                                  preferred_element_type=jnp.float32)
        out_ref[CODE:] = jnp.zeros((_P1_ROWS - CODE, N), jnp.float32)


def _build_t0(x_t, ccl0_t):
    return pl.pallas_call(
        _p1_body,
        grid=(_P1_ENC_BLOCKS + 1,),
        in_specs=[
            pl.BlockSpec((_P1_PIX, B),
                         lambda i: (jnp.minimum(i, _P1_ENC_BLOCKS - 1), 0)),
            pl.BlockSpec((CODE, C), lambda i: (0, 0)),
        ],
        out_specs=pl.BlockSpec((_P1_ROWS, N), lambda i: (i, 0)),
        out_shape=jax.ShapeDtypeStruct((T0_ROWS, N), jnp.float32),
    )(x_t, ccl0_t)


_P2_BLK = 1024
_P2_GRID = W // _P2_BLK


def _p2_body(lg_ref, c0_ref, c1_ref, c2_ref):
    om = 0.5 + 0.5 * jnp.sin(lg_ref[...])
    ji = lax.broadcasted_iota(jnp.int32, (4, 64), 0)
    ci = lax.broadcasted_iota(jnp.int32, (4, 64), 1)
    sel = (ci // 16 == ji).astype(jnp.float32)
    for li, out_ref in enumerate((c0_ref, c1_ref, c2_ref)):
        w00 = om[li, 0:1, :]
        w01 = om[li, 1:2, :]
        w10 = om[li, 2:3, :]
        w11 = om[li, 3:4, :]
        terms = jnp.concatenate(
            [w00, w10 - w00, w01 - w00, w11 - w10 - w01 + w00], axis=0)
        out_ref[...] = lax.dot_general(
            terms, sel, (((0,), (0,)), ((), ())),
            preferred_element_type=jnp.float32)


def _build_coefs(logits_t):
    return pl.pallas_call(
        _p2_body,
        grid=(_P2_GRID,),
        in_specs=[pl.BlockSpec((3, 4, _P2_BLK), lambda i: (0, 0, i))],
        out_specs=[pl.BlockSpec((_P2_BLK, 64), lambda i: (i, 0))] * 3,
        out_shape=[jax.ShapeDtypeStruct((W, 64), jnp.float32)] * 3,
    )(logits_t)


def _p3_body(cc_ref, codes_ref):
    code = 0.5 + 0.5 * jnp.sin(cc_ref[...])
    ind = _class_indicator()
    for j in range(2):
        codes_ref[j] = jnp.dot(code[j], ind,
                               preferred_element_type=jnp.float32)


def _build_codes(ccl12_t):
    return pl.pallas_call(
        _p3_body,
        out_shape=jax.ShapeDtypeStruct((2, CODE, N), jnp.float32),
    )(ccl12_t)


_P4_BLK = 2048
_P4_GRID = W // _P4_BLK


def _p4_body(v_ref, out_ref):
    i = pl.program_id(0)
    s = jnp.sum(v_ref[...], axis=0, keepdims=True)

    @pl.when(i == 0)
    def _():
        out_ref[...] = s

    @pl.when(i > 0)
    def _():
        out_ref[...] += s


def _column_sum(v3):
    return pl.pallas_call(
        _p4_body,
        grid=(_P4_GRID,),
        in_specs=[pl.BlockSpec((_P4_BLK, N), lambda i: (i, 0))],
        out_specs=pl.BlockSpec((1, N), lambda i: (0, 0)),
        out_shape=jax.ShapeDtypeStruct((1, N), jnp.float32),
    )(v3)


_MESH = plsc.VectorSubcoreMesh(core_axis_name="c", subcore_axis_name="s",
                               num_cores=NCORES, num_subcores=NSUB)


def _base_scratch(extra):
    return [
        pltpu.VMEM((2 * ROWS_PER_TILE,), jnp.int32),
        pltpu.VMEM((K, 64), jnp.float32),
        pltpu.VMEM((2 * K, N), jnp.float32),
        pltpu.VMEM((K, N), jnp.float32),
    ] + extra + [
        pltpu.SemaphoreType.DMA,
        pltpu.SemaphoreType.DMA,
    ]


def _worker_id():
    return lax.axis_index("s") * NCORES + lax.axis_index("c")


def _gather_chunk(ci, base0, table, coef, idx_v, coef_v, lr_v, sem1):
    loc = pl.ds(ci * 2 * K, 2 * K)
    cp = pltpu.async_copy(table.at[idx_v.at[loc]], lr_v, sem1)
    pltpu.sync_copy(coef.at[pl.ds(base0 + ci * K, K)], coef_v)
    cp.wait()


def _coef_vecs(coef_v, r):
    return [coef_v[r, pl.ds(16 * j, 16)] for j in range(4)]


def _layer_body(table, lridx, coef, codes, out,
                idx_v, coef_v, lr_v, o_v, sem1, sem2):
    wid = _worker_id()
    base0 = wid * ROWS_PER_TILE
    pltpu.sync_copy(lridx.at[pl.ds(2 * base0, 2 * ROWS_PER_TILE)], idx_v)

    def chunk(ci, carry):
        _gather_chunk(ci, base0, table, coef, idx_v, coef_v, lr_v, sem1)

        @pl.when(ci > 0)
        def _():
            pltpu.make_async_copy(o_v, out.at[pl.ds(base0, K)], sem2).wait()

        def row(r, carry2):
            a, b, c, d = _coef_vecs(coef_v, r)
            for cb in range(N // 16):
                sl = pl.ds(cb * 16, 16)
                l = lr_v[r, sl]
                rr = lr_v[K + r, sl]
                o_v[r, sl] = a + b * l + c * rr + d * (l * rr)
            return carry2

        lax.fori_loop(0, K, row, 0)
        pltpu.async_copy(o_v, out.at[pl.ds(base0 + ci * K, K)], sem2)
        return carry

    lax.fori_loop(0, CHUNKS, chunk, 0)
    pltpu.make_async_copy(o_v, out.at[pl.ds(base0, K)], sem2).wait()
    if codes is not None:
        cb = wid * CODE_PER_TILE
        stage = lr_v.at[pl.ds(0, CODE_PER_TILE)]
        pltpu.sync_copy(codes.at[pl.ds(cb, CODE_PER_TILE)], stage)
        pltpu.sync_copy(stage, out.at[pl.ds(W + cb, CODE_PER_TILE)])


def _mid_layer(table, lridx, coef, codes):
    return pl.kernel(
        _layer_body,
        out_type=jax.ShapeDtypeStruct((T_ROWS, N), jnp.float32),
        mesh=_MESH,
        scratch_types=_base_scratch([]),
    )(table, lridx, coef, codes)


def _final_layer_body(table, lridx, coef, out,
                      idx_v, coef_v, lr_v, o_v, sem1, sem2):
    _layer_body(table, lridx, coef, None, out,
                idx_v, coef_v, lr_v, o_v, sem1, sem2)


def _final_layer(table, lridx, coef):
    return pl.kernel(
        _final_layer_body,
        out_type=jax.ShapeDtypeStruct((W, N), jnp.float32),
        mesh=_MESH,
        scratch_types=_base_scratch([]),
    )(table, lridx, coef)


def _interleave_idx(left, right):
    l2 = left.reshape(W // K, K)
    r2 = right.reshape(W // K, K)
    return jnp.concatenate([l2, r2], axis=1).reshape(2 * W)


def kernel(x, logits0, logits1, logits2, class_code_logits,
           left0, right0, left1, right1, left2, right2):
    x_t = x.reshape(B, -1).T
    ccl0_t = class_code_logits[0].T
    ccl12_t = jnp.transpose(class_code_logits[1:], (0, 2, 1))
    logits_t = jnp.transpose(
        jnp.stack([logits0, logits1, logits2]), (0, 2, 1))

    t0 = _build_t0(x_t, ccl0_t)
    coef0, coef1, coef2 = _build_coefs(logits_t)
    codes = _build_codes(ccl12_t)

    t1 = _mid_layer(t0, _interleave_idx(left0, right0), coef0, codes[0])
    t2 = _mid_layer(t1, _interleave_idx(left1, right1), coef1, codes[1])
    v3 = _final_layer(t2, _interleave_idx(left2, right2), coef2)
    sums = _column_sum(v3)

    res = sums[0] / TAU
    return res.reshape(C, B).T


# speedup vs baseline: 1.2727x; 1.0573x over previous
import jax
import jax.numpy as jnp
from jax import lax
from jax.experimental import pallas as pl
from jax.experimental.pallas import tpu as pltpu
from jax.experimental.pallas import tpu_sc as plsc

B = 128
C = 10
N = B * C
W = 16384
ENC = 12288
CODE = 128
T0_ROWS = ENC + CODE
T_ROWS = W + CODE
TAU = 100.0

NCORES = 2
NSUB = 16
NW = NCORES * NSUB
ROWS_PER_TILE = W // NW
K = 32
CHUNKS = ROWS_PER_TILE // K
CODE_PER_TILE = CODE // NW


def _class_indicator():
    ci = lax.broadcasted_iota(jnp.int32, (C, N), 0)
    ni = lax.broadcasted_iota(jnp.int32, (C, N), 1)
    return (ni // B == ci).astype(jnp.float32)


_P1_PIX = 512
_P1_ROWS = 4 * _P1_PIX
_P1_ENC_BLOCKS = ENC // _P1_ROWS


def _p1_body(xt_ref, c0_ref, out_ref):
    i = pl.program_id(0)

    @pl.when(i < _P1_ENC_BLOCKS)
    def _():
        xb = xt_ref[...]
        bits = [(xb > (t + 1) / 5.0).astype(jnp.float32) for t in range(4)]
        bits = jnp.stack(bits, axis=1).reshape(_P1_ROWS, B)
        out_ref[...] = pltpu.repeat(bits, C, axis=1)

    @pl.when(i == _P1_ENC_BLOCKS)
    def _():
        code = 0.5 + 0.5 * jnp.sin(c0_ref[...])
        out_ref[0:CODE] = jnp.dot(code, _class_indicator(),
                                  preferred_element_type=jnp.float32)
        out_ref[CODE:] = jnp.zeros((_P1_ROWS - CODE, N), jnp.float32)


def _build_t0(x_t, ccl0_t):
    return pl.pallas_call(
        _p1_body,
        grid=(_P1_ENC_BLOCKS + 1,),
        in_specs=[
            pl.BlockSpec((_P1_PIX, B),
                         lambda i: (jnp.minimum(i, _P1_ENC_BLOCKS - 1), 0)),
            pl.BlockSpec((CODE, C), lambda i: (0, 0)),
        ],
        out_specs=pl.BlockSpec((_P1_ROWS, N), lambda i: (i, 0)),
        out_shape=jax.ShapeDtypeStruct((T0_ROWS, N), jnp.float32),
    )(x_t, ccl0_t)


_P2_BLK = 1024
_P2_GRID = W // _P2_BLK


def _p2_body(lg_ref, c0_ref, c1_ref, c2_ref):
    om = 0.5 + 0.5 * jnp.sin(lg_ref[...])
    ji = lax.broadcasted_iota(jnp.int32, (4, 64), 0)
    ci = lax.broadcasted_iota(jnp.int32, (4, 64), 1)
    sel = (ci // 16 == ji).astype(jnp.float32)
    for li, out_ref in enumerate((c0_ref, c1_ref, c2_ref)):
        w00 = om[li, 0:1, :]
        w01 = om[li, 1:2, :]
        w10 = om[li, 2:3, :]
        w11 = om[li, 3:4, :]
        terms = jnp.concatenate(
            [w00, w10 - w00, w01 - w00, w11 - w10 - w01 + w00], axis=0)
        out_ref[...] = lax.dot_general(
            terms, sel, (((0,), (0,)), ((), ())),
            preferred_element_type=jnp.float32)


def _build_coefs(logits_t):
    return pl.pallas_call(
        _p2_body,
        grid=(_P2_GRID,),
        in_specs=[pl.BlockSpec((3, 4, _P2_BLK), lambda i: (0, 0, i))],
        out_specs=[pl.BlockSpec((_P2_BLK, 64), lambda i: (i, 0))] * 3,
        out_shape=[jax.ShapeDtypeStruct((W, 64), jnp.float32)] * 3,
    )(logits_t)


def _p3_body(cc_ref, codes_ref):
    code = 0.5 + 0.5 * jnp.sin(cc_ref[...])
    ind = _class_indicator()
    for j in range(2):
        codes_ref[j] = jnp.dot(code[j], ind,
                               preferred_element_type=jnp.float32)


def _build_codes(ccl12_t):
    return pl.pallas_call(
        _p3_body,
        out_shape=jax.ShapeDtypeStruct((2, CODE, N), jnp.float32),
    )(ccl12_t)


_P4_BLK = 2048
_P4_GRID = W // _P4_BLK


def _p4_body(v_ref, out_ref):
    i = pl.program_id(0)
    s = jnp.sum(v_ref[...], axis=0, keepdims=True)

    @pl.when(i == 0)
    def _():
        out_ref[...] = s

    @pl.when(i > 0)
    def _():
        out_ref[...] += s


def _column_sum(v3):
    return pl.pallas_call(
        _p4_body,
        grid=(_P4_GRID,),
        in_specs=[pl.BlockSpec((_P4_BLK, N), lambda i: (i, 0))],
        out_specs=pl.BlockSpec((1, N), lambda i: (0, 0)),
        out_shape=jax.ShapeDtypeStruct((1, N), jnp.float32),
    )(v3)


_MESH = plsc.VectorSubcoreMesh(core_axis_name="c", subcore_axis_name="s",
                               num_cores=NCORES, num_subcores=NSUB)


def _base_scratch(extra):
    return [
        pltpu.VMEM((2 * ROWS_PER_TILE,), jnp.int32),
        pltpu.VMEM((K, 64), jnp.float32),
        pltpu.VMEM((2 * K, N), jnp.float32),
        pltpu.VMEM((K, N), jnp.float32),
    ] + extra + [
        pltpu.SemaphoreType.DMA,
        pltpu.SemaphoreType.DMA,
    ]


def _worker_id():
    return lax.axis_index("s") * NCORES + lax.axis_index("c")


def _gather_chunk(ci, base0, table, coef, idx_v, coef_v, lr_v, sem1):
    loc = pl.ds(ci * 2 * K, 2 * K)
    cp = pltpu.async_copy(table.at[idx_v.at[loc]], lr_v, sem1)
    pltpu.sync_copy(coef.at[pl.ds(base0 + ci * K, K)], coef_v)
    cp.wait()


def _coef_vecs(coef_v, r):
    return [coef_v[r, pl.ds(16 * j, 16)] for j in range(4)]


def _layer_body(table, lridx, coef, codes, out,
                idx_v, coef_v, lr_v, o_v, sem1, sem2):
    wid = _worker_id()
    base0 = wid * ROWS_PER_TILE
    pltpu.sync_copy(lridx.at[pl.ds(2 * base0, 2 * ROWS_PER_TILE)], idx_v)

    def chunk(ci, carry):
        _gather_chunk(ci, base0, table, coef, idx_v, coef_v, lr_v, sem1)

        @pl.when(ci > 0)
        def _():
            pltpu.make_async_copy(o_v, out.at[pl.ds(base0, K)], sem2).wait()

        def row(r, carry2):
            a, b, c, d = _coef_vecs(coef_v, r)
            for cb in range(N // 16):
                sl = pl.ds(cb * 16, 16)
                l = lr_v[r, sl]
                rr = lr_v[K + r, sl]
                o_v[r, sl] = a + c * rr + l * (b + d * rr)
            return carry2

        lax.fori_loop(0, K, row, 0)
        pltpu.async_copy(o_v, out.at[pl.ds(base0 + ci * K, K)], sem2)
        return carry

    lax.fori_loop(0, CHUNKS, chunk, 0)
    pltpu.make_async_copy(o_v, out.at[pl.ds(base0, K)], sem2).wait()
    if codes is not None:
        cb = wid * CODE_PER_TILE
        stage = lr_v.at[pl.ds(0, CODE_PER_TILE)]
        pltpu.sync_copy(codes.at[pl.ds(cb, CODE_PER_TILE)], stage)
        pltpu.sync_copy(stage, out.at[pl.ds(W + cb, CODE_PER_TILE)])


def _mid_layer(table, lridx, coef, codes):
    return pl.kernel(
        _layer_body,
        out_type=jax.ShapeDtypeStruct((T_ROWS, N), jnp.float32),
        mesh=_MESH,
        scratch_types=_base_scratch([]),
    )(table, lridx, coef, codes)


def _final_layer_body(table, lridx, coef, out,
                      idx_v, coef_v, lr_v, o_v, sem1, sem2):
    _layer_body(table, lridx, coef, None, out,
                idx_v, coef_v, lr_v, o_v, sem1, sem2)


def _final_layer(table, lridx, coef):
    return pl.kernel(
        _final_layer_body,
        out_type=jax.ShapeDtypeStruct((W, N), jnp.float32),
        mesh=_MESH,
        scratch_types=_base_scratch([]),
    )(table, lridx, coef)


def _interleave_idx(left, right):
    l2 = left.reshape(W // K, K)
    r2 = right.reshape(W // K, K)
    return jnp.concatenate([l2, r2], axis=1).reshape(2 * W)


def kernel(x, logits0, logits1, logits2, class_code_logits,
           left0, right0, left1, right1, left2, right2):
    x_t = x.reshape(B, -1).T
    ccl0_t = class_code_logits[0].T
    ccl12_t = jnp.transpose(class_code_logits[1:], (0, 2, 1))
    logits_t = jnp.transpose(
        jnp.stack([logits0, logits1, logits2]), (0, 2, 1))

    t0 = _build_t0(x_t, ccl0_t)
    coef0, coef1, coef2 = _build_coefs(logits_t)
    codes = _build_codes(ccl12_t)

    t1 = _mid_layer(t0, _interleave_idx(left0, right0), coef0, codes[0])
    t2 = _mid_layer(t1, _interleave_idx(left1, right1), coef1, codes[1])
    v3 = _final_layer(t2, _interleave_idx(left2, right2), coef2)
    sums = _column_sum(v3)

    res = sums[0] / TAU
    return res.reshape(C, B).T
